# baseline (speedup 1.0000x reference)
import functools

import jax
import jax.numpy as jnp
from jax import lax
from jax.experimental import pallas as pl
from jax.experimental.pallas import tpu as pltpu
from jax.experimental.pallas import tpu_sc as plsc

E = 64
D = 768
H = 4 * D
T = 2048
TILE = 128
W = E + T // TILE
NROWS = W * TILE
LB_COEF = 0.01
NWORKERS = 32
CH = T // NWORKERS


def _gate_route_body(x_ref, wg_ref, bg_ref, slot_ref, tab_ref, loss_ref):
    x = x_ref[...]
    logits = jnp.dot(x, wg_ref[...], preferred_element_type=jnp.float32)
    logits = logits + bg_ref[...]

    m = jnp.max(logits, axis=1, keepdims=True)
    p = jnp.exp(logits - m)
    s = jnp.sum(p, axis=1, keepdims=True)
    soft = p / s

    usage = jnp.sum(soft, axis=0, keepdims=True) * (1.0 / T)
    diff = usage - (1.0 / E)
    loss_ref[...] = (LB_COEF / E) * jnp.sum(diff * diff, axis=1, keepdims=True)

    col = lax.broadcasted_iota(jnp.int32, (T, E), 1)
    eid = jnp.min(jnp.where(logits == m, col, E), axis=1, keepdims=True)

    onehot = (eid == col).astype(jnp.float32)
    counts = jnp.sum(onehot, axis=0, keepdims=True)
    tiles = jnp.ceil(counts * (1.0 / TILE))
    padded = tiles * TILE

    r64 = lax.broadcasted_iota(jnp.int32, (E, E), 0)
    c64 = lax.broadcasted_iota(jnp.int32, (E, E), 1)
    incl = (r64 <= c64).astype(jnp.float32)
    cumpad = jnp.dot(padded, incl, preferred_element_type=jnp.float32)
    cumtiles = jnp.dot(tiles, incl, preferred_element_type=jnp.float32)
    pbase = cumpad - padded

    rT = lax.broadcasted_iota(jnp.int32, (T, T), 0)
    cT = lax.broadcasted_iota(jnp.int32, (T, T), 1)
    lower = (rT > cT).astype(jnp.float32)
    csum = jnp.dot(lower, onehot, preferred_element_type=jnp.float32)
    rank = jnp.sum(onehot * csum, axis=1, keepdims=True)

    pb_tok = jnp.sum(onehot * pbase, axis=1, keepdims=True)
    slot_ref[...] = (pb_tok + rank).astype(jnp.int32)

    tab_ref[...] = jnp.concatenate([cumtiles - tiles, tiles],
                                   axis=0).astype(jnp.int32)


def _gate_route(x2d, Wg, bg2d):
    return pl.pallas_call(
        _gate_route_body,
        out_shape=(
            jax.ShapeDtypeStruct((T, 1), jnp.int32),
            jax.ShapeDtypeStruct((2, E), jnp.int32),
            jax.ShapeDtypeStruct((1, 1), jnp.float32),
        ),
    )(x2d, Wg, bg2d)


def _ffn_body(tab_ref, xg_hbm, w1_ref, b1_ref, w2_ref, b2_ref, buf_hbm,
              xin, ob, sem_in, sem_out):
    e = pl.program_id(0)
    base_t = tab_ref[0, e]
    nt = tab_ref[1, e]
    w1 = w1_ref[0].astype(jnp.bfloat16)
    w2 = w2_ref[0].astype(jnp.bfloat16)

    def tile_step(t, carry):
        row0 = (base_t + t) * TILE
        pltpu.make_async_copy(
            xg_hbm.at[pl.ds(row0, TILE)], xin, sem_in).start()
        pltpu.make_async_copy(
            xg_hbm.at[pl.ds(row0, TILE)], xin, sem_in).wait()
        h = jnp.dot(xin[...].astype(jnp.bfloat16), w1,
                    preferred_element_type=jnp.float32)
        h = h + b1_ref[0]
        h = 0.5 * h * (1.0 + lax.erf(h * 0.7071067811865476))
        o = jnp.dot(h.astype(jnp.bfloat16), w2,
                    preferred_element_type=jnp.float32)
        ob[...] = o + b2_ref[0]
        pltpu.make_async_copy(
            ob, buf_hbm.at[pl.ds(row0, TILE)], sem_out).start()
        pltpu.make_async_copy(
            ob, buf_hbm.at[pl.ds(row0, TILE)], sem_out).wait()
        return carry

    lax.fori_loop(0, nt, tile_step, 0)


def _ffn(tab, xg, W1, b1r, W2, b2r):
    return pl.pallas_call(
        _ffn_body,
        grid=(E,),
        in_specs=[
            pl.BlockSpec(memory_space=pltpu.SMEM),
            pl.BlockSpec(memory_space=pl.ANY),
            pl.BlockSpec((1, D, H), lambda e: (e, 0, 0)),
            pl.BlockSpec((1, 1, H), lambda e: (e, 0, 0)),
            pl.BlockSpec((1, H, D), lambda e: (e, 0, 0)),
            pl.BlockSpec((1, 1, D), lambda e: (e, 0, 0)),
        ],
        out_specs=pl.BlockSpec(memory_space=pl.ANY),
        out_shape=jax.ShapeDtypeStruct((NROWS, D), jnp.float32),
        scratch_shapes=[
            pltpu.VMEM((TILE, D), jnp.float32),
            pltpu.VMEM((TILE, D), jnp.float32),
            pltpu.SemaphoreType.DMA,
            pltpu.SemaphoreType.DMA,
        ],
        compiler_params=pltpu.CompilerParams(
            dimension_semantics=("arbitrary",),
        ),
    )(tab, xg, W1, b1r, W2, b2r)


def _sc_wid():
    return lax.axis_index("s") * 2 + lax.axis_index("c")


def _scatter_body(x_hbm, slot_hbm, xg_hbm, idx_v, rows_v, sem):
    base = _sc_wid() * CH
    pltpu.sync_copy(slot_hbm.at[pl.ds(base, CH)], idx_v)
    pltpu.sync_copy(x_hbm.at[pl.ds(base, CH)], rows_v)
    pltpu.async_copy(rows_v, xg_hbm.at[idx_v], sem).wait()


def _scatter(x2d, slot1d):
    mesh = plsc.VectorSubcoreMesh(core_axis_name="c", subcore_axis_name="s")
    f = pl.kernel(
        _scatter_body,
        out_type=jax.ShapeDtypeStruct((NROWS, D), jnp.float32),
        mesh=mesh,
        scratch_types=[
            pltpu.VMEM((CH,), jnp.int32),
            pltpu.VMEM((CH, D), jnp.float32),
            pltpu.SemaphoreType.DMA,
        ],
    )
    return f(x2d, slot1d)


def _unperm_body(buf_hbm, slot_hbm, out_hbm, idx_v, rows_v, sem):
    base = _sc_wid() * CH
    pltpu.sync_copy(slot_hbm.at[pl.ds(base, CH)], idx_v)
    pltpu.async_copy(buf_hbm.at[idx_v], rows_v, sem).wait()
    pltpu.sync_copy(rows_v, out_hbm.at[pl.ds(base, CH)])


def _unperm(buf, slot1d):
    mesh = plsc.VectorSubcoreMesh(core_axis_name="c", subcore_axis_name="s")
    f = pl.kernel(
        _unperm_body,
        out_type=jax.ShapeDtypeStruct((T, D), jnp.float32),
        mesh=mesh,
        scratch_types=[
            pltpu.VMEM((CH,), jnp.int32),
            pltpu.VMEM((CH, D), jnp.float32),
            pltpu.SemaphoreType.DMA,
        ],
    )
    return f(buf, slot1d)


def kernel(x, Wg, bg, W1, b1, W2, b2):
    Bs, Ts, C = x.shape
    x2d = x.reshape(T, D)
    slot2d, tab, loss = _gate_route(x2d, Wg, bg.reshape(1, E))
    slot = slot2d.reshape(T)
    xg = _scatter(x2d, slot)
    buf = _ffn(tab, xg, W1, b1.reshape(E, 1, H), W2, b2.reshape(E, 1, D))
    out2d = _unperm(buf, slot)
    return out2d.reshape(Bs, Ts, C), loss[0, 0]


# speedup vs baseline: 1.3861x; 1.3861x over previous
import functools

import jax
import jax.numpy as jnp
from jax import lax
from jax.experimental import pallas as pl
from jax.experimental.pallas import tpu as pltpu
from jax.experimental.pallas import tpu_sc as plsc

E = 64
D = 768
H = 4 * D
T = 2048
TILE = 128
W = E + T // TILE
NROWS = W * TILE
LB_COEF = 0.01
NWORKERS = 32
CH = T // NWORKERS


def _gate_route_body(x_ref, wg_ref, bg_ref, slot_ref, ew_ref, loss_ref):
    x = x_ref[...]
    logits = jnp.dot(x, wg_ref[...], preferred_element_type=jnp.float32)
    logits = logits + bg_ref[...]

    m = jnp.max(logits, axis=1, keepdims=True)
    p = jnp.exp(logits - m)
    s = jnp.sum(p, axis=1, keepdims=True)
    soft = p / s

    usage = jnp.sum(soft, axis=0, keepdims=True) * (1.0 / T)
    diff = usage - (1.0 / E)
    loss_ref[...] = (LB_COEF / E) * jnp.sum(diff * diff, axis=1, keepdims=True)

    col = lax.broadcasted_iota(jnp.int32, (T, E), 1)
    eid = jnp.min(jnp.where(logits == m, col, E), axis=1, keepdims=True)

    onehot = (eid == col).astype(jnp.float32)
    counts = jnp.sum(onehot, axis=0, keepdims=True)
    tiles = jnp.ceil(counts * (1.0 / TILE))
    padded = tiles * TILE

    r64 = lax.broadcasted_iota(jnp.int32, (E, E), 0)
    c64 = lax.broadcasted_iota(jnp.int32, (E, E), 1)
    incl = (r64 <= c64).astype(jnp.float32)
    cumpad = jnp.dot(padded, incl, preferred_element_type=jnp.float32)
    cumtiles = jnp.dot(tiles, incl, preferred_element_type=jnp.float32)
    pbase = cumpad - padded

    rT = lax.broadcasted_iota(jnp.int32, (T, T), 0)
    cT = lax.broadcasted_iota(jnp.int32, (T, T), 1)
    lower = (rT > cT).astype(jnp.float32)
    csum = jnp.dot(lower, onehot, preferred_element_type=jnp.float32)
    rank = jnp.sum(onehot * csum, axis=1, keepdims=True)

    pb_tok = jnp.sum(onehot * pbase, axis=1, keepdims=True)
    slot_ref[...] = (pb_tok + rank).astype(jnp.int32)

    wrow = lax.broadcasted_iota(jnp.int32, (W, E), 0).astype(jnp.float32)
    ge = (cumtiles <= wrow).astype(jnp.float32)
    ew = jnp.sum(ge, axis=1, keepdims=True).astype(jnp.int32)
    ew_ref[...] = jnp.minimum(ew, E - 1)


def _gate_route(x2d, Wg, bg2d):
    return pl.pallas_call(
        _gate_route_body,
        out_shape=(
            jax.ShapeDtypeStruct((T, 1), jnp.int32),
            jax.ShapeDtypeStruct((W, 1), jnp.int32),
            jax.ShapeDtypeStruct((1, 1), jnp.float32),
        ),
    )(x2d, Wg, bg2d)


def _ffn_body(ew_ref, xg_ref, w1_ref, b1_ref, w2_ref, b2_ref, o_ref):
    e = ew_ref[pl.program_id(0)]
    xb = xg_ref[...].astype(jnp.bfloat16)
    h = jnp.dot(xb, w1_ref[0].astype(jnp.bfloat16),
                preferred_element_type=jnp.float32)
    h = h + b1_ref[pl.ds(e, 1), :]
    h = 0.5 * h * (1.0 + lax.erf(h * 0.7071067811865476))
    o = jnp.dot(h.astype(jnp.bfloat16), w2_ref[0].astype(jnp.bfloat16),
                preferred_element_type=jnp.float32)
    o_ref[...] = o + b2_ref[pl.ds(e, 1), :]


def _ffn(ew, xg, W1, b1, W2, b2):
    grid_spec = pltpu.PrefetchScalarGridSpec(
        num_scalar_prefetch=1,
        grid=(W,),
        in_specs=[
            pl.BlockSpec((TILE, D), lambda w, ew: (w, 0)),
            pl.BlockSpec((1, D, H), lambda w, ew: (ew[w], 0, 0)),
            pl.BlockSpec((E, H), lambda w, ew: (0, 0)),
            pl.BlockSpec((1, H, D), lambda w, ew: (ew[w], 0, 0)),
            pl.BlockSpec((E, D), lambda w, ew: (0, 0)),
        ],
        out_specs=pl.BlockSpec((TILE, D), lambda w, ew: (w, 0)),
    )
    return pl.pallas_call(
        _ffn_body,
        grid_spec=grid_spec,
        out_shape=jax.ShapeDtypeStruct((NROWS, D), jnp.float32),
        compiler_params=pltpu.CompilerParams(
            dimension_semantics=("arbitrary",),
        ),
    )(ew, xg, W1, b1, W2, b2)


def _sc_wid():
    return lax.axis_index("s") * 2 + lax.axis_index("c")


def _scatter_body(x_hbm, slot_hbm, xg_hbm, idx_v, rows_v, sem):
    base = _sc_wid() * CH
    pltpu.sync_copy(slot_hbm.at[pl.ds(base, CH)], idx_v)
    pltpu.sync_copy(x_hbm.at[pl.ds(base, CH)], rows_v)
    pltpu.async_copy(rows_v, xg_hbm.at[idx_v], sem).wait()


def _scatter(x2d, slot1d):
    mesh = plsc.VectorSubcoreMesh(core_axis_name="c", subcore_axis_name="s")
    f = pl.kernel(
        _scatter_body,
        out_type=jax.ShapeDtypeStruct((NROWS, D), jnp.float32),
        mesh=mesh,
        scratch_types=[
            pltpu.VMEM((CH,), jnp.int32),
            pltpu.VMEM((CH, D), jnp.float32),
            pltpu.SemaphoreType.DMA,
        ],
    )
    return f(x2d, slot1d)


def _unperm_body(buf_hbm, slot_hbm, out_hbm, idx_v, rows_v, sem):
    base = _sc_wid() * CH
    pltpu.sync_copy(slot_hbm.at[pl.ds(base, CH)], idx_v)
    pltpu.async_copy(buf_hbm.at[idx_v], rows_v, sem).wait()
    pltpu.sync_copy(rows_v, out_hbm.at[pl.ds(base, CH)])


def _unperm(buf, slot1d):
    mesh = plsc.VectorSubcoreMesh(core_axis_name="c", subcore_axis_name="s")
    f = pl.kernel(
        _unperm_body,
        out_type=jax.ShapeDtypeStruct((T, D), jnp.float32),
        mesh=mesh,
        scratch_types=[
            pltpu.VMEM((CH,), jnp.int32),
            pltpu.VMEM((CH, D), jnp.float32),
            pltpu.SemaphoreType.DMA,
        ],
    )
    return f(buf, slot1d)


def kernel(x, Wg, bg, W1, b1, W2, b2):
    Bs, Ts, C = x.shape
    x2d = x.reshape(T, D)
    slot2d, ew2d, loss = _gate_route(x2d, Wg, bg.reshape(1, E))
    slot = slot2d.reshape(T)
    ew = ew2d.reshape(W)
    xg = _scatter(x2d, slot)
    buf = _ffn(ew, xg, W1, b1, W2, b2)
    out2d = _unperm(buf, slot)
    return out2d.reshape(Bs, Ts, C), loss[0, 0]
